# Initial kernel scaffold; baseline (speedup 1.0000x reference)
import jax
import jax.numpy as jnp
from jax.experimental import pallas as pl


def kernel(token_hidden, router_logits, expert_ground, alpha):
    raise NotImplementedError("write your pallas kernel here")


# speedup vs baseline: 1.4744x; 1.4744x over previous
import functools

import jax
import jax.numpy as jnp
from jax.experimental import pallas as pl
from jax.experimental.pallas import tpu as pltpu


def _router_kernel(x_ref, r_ref, egt_ref, o_ref):
    x = x_ref[...]
    egt = egt_ref[...]
    dots = jnp.dot(x, egt, preferred_element_type=jnp.float32)
    scores = r_ref[...] + dots

    e_dim = scores.shape[1]
    col = jax.lax.broadcasted_iota(jnp.int32, scores.shape, 1)

    m1 = jnp.max(scores, axis=1, keepdims=True)
    i1 = jnp.min(jnp.where(scores == m1, col, e_dim), axis=1, keepdims=True)
    masked = jnp.where(col == i1, -jnp.inf, scores)
    m2 = jnp.max(masked, axis=1, keepdims=True)
    i2 = jnp.min(jnp.where(masked == m2, col, e_dim), axis=1, keepdims=True)

    e = jnp.exp(m2 - m1)
    s = 1.0 + e
    w0 = 1.0 / s
    w1 = e / s
    o_ref[...] = jnp.concatenate(
        [i1.astype(jnp.float32), w0, i2.astype(jnp.float32), w1], axis=1
    )


def kernel(token_hidden, router_logits, expert_ground, alpha):
    T, D = token_hidden.shape
    E = expert_ground.shape[0]
    TB = 512
    egt = jnp.float32(alpha) * expert_ground.T

    out = pl.pallas_call(
        _router_kernel,
        grid=(T // TB,),
        in_specs=[
            pl.BlockSpec((TB, D), lambda i: (i, 0)),
            pl.BlockSpec((TB, E), lambda i: (i, 0)),
            pl.BlockSpec((D, E), lambda i: (0, 0)),
        ],
        out_specs=pl.BlockSpec((TB, 4), lambda i: (i, 0)),
        out_shape=jax.ShapeDtypeStruct((T, 4), jnp.float32),
        compiler_params=pltpu.CompilerParams(
            dimension_semantics=("arbitrary",),
        ),
    )(token_hidden, router_logits, egt)

    return out.reshape(T, 2, 2)


# speedup vs baseline: 1.4763x; 1.0013x over previous
import functools

import jax
import jax.numpy as jnp
from jax.experimental import pallas as pl
from jax.experimental.pallas import tpu as pltpu


def _router_kernel(x_ref, r_ref, egt_ref, o_ref):
    x = x_ref[...]
    egt = egt_ref[...]
    dots = jnp.dot(x, egt, preferred_element_type=jnp.float32)
    scores = r_ref[...] + dots

    e_dim = scores.shape[1]
    col = jax.lax.broadcasted_iota(jnp.int32, scores.shape, 1)

    m1 = jnp.max(scores, axis=1, keepdims=True)
    i1 = jnp.min(jnp.where(scores == m1, col, e_dim), axis=1, keepdims=True)
    masked = jnp.where(col == i1, -jnp.inf, scores)
    m2 = jnp.max(masked, axis=1, keepdims=True)
    i2 = jnp.min(jnp.where(masked == m2, col, e_dim), axis=1, keepdims=True)

    e = jnp.exp(m2 - m1)
    s = 1.0 + e
    w0 = 1.0 / s
    w1 = e / s
    o_ref[...] = jnp.concatenate(
        [i1.astype(jnp.float32), w0, i2.astype(jnp.float32), w1], axis=1
    )


def kernel(token_hidden, router_logits, expert_ground, alpha):
    T, D = token_hidden.shape
    E = expert_ground.shape[0]
    TB = 512
    egt = jnp.float32(alpha) * expert_ground.T

    out = pl.pallas_call(
        _router_kernel,
        grid=(T // TB,),
        in_specs=[
            pl.BlockSpec((TB, D), lambda i: (i, 0)),
            pl.BlockSpec((TB, E), lambda i: (i, 0)),
            pl.BlockSpec((D, E), lambda i: (0, 0)),
        ],
        out_specs=pl.BlockSpec((TB, 4), lambda i: (i, 0)),
        out_shape=jax.ShapeDtypeStruct((T, 4), jnp.float32),
        compiler_params=pltpu.CompilerParams(
            dimension_semantics=("parallel",),
        ),
    )(token_hidden, router_logits, egt)

    return out.reshape(T, 2, 2)


# speedup vs baseline: 1.7110x; 1.1589x over previous
import functools

import jax
import jax.numpy as jnp
from jax.experimental import pallas as pl
from jax.experimental.pallas import tpu as pltpu


def _router_kernel(x_ref, r_ref, egt_ref, o_ref):
    x = x_ref[...]
    egt = egt_ref[...]
    dots = jnp.dot(x, egt, preferred_element_type=jnp.float32)
    scores = r_ref[...] + dots

    e_dim = scores.shape[1]
    col = jax.lax.broadcasted_iota(jnp.int32, scores.shape, 1)

    m1 = jnp.max(scores, axis=1, keepdims=True)
    i1 = jnp.min(jnp.where(scores == m1, col, e_dim), axis=1, keepdims=True)
    masked = jnp.where(col == i1, -jnp.inf, scores)
    m2 = jnp.max(masked, axis=1, keepdims=True)
    i2 = jnp.min(jnp.where(masked == m2, col, e_dim), axis=1, keepdims=True)

    e = jnp.exp(m2 - m1)
    s = 1.0 + e
    w0 = 1.0 / s
    w1 = e / s
    o_ref[...] = jnp.concatenate(
        [i1.astype(jnp.float32), w0, i2.astype(jnp.float32), w1], axis=1
    )


def kernel(token_hidden, router_logits, expert_ground, alpha):
    T, D = token_hidden.shape
    E = expert_ground.shape[0]
    TB = 1024
    egt = jnp.float32(alpha) * expert_ground.T

    out = pl.pallas_call(
        _router_kernel,
        grid=(T // TB,),
        in_specs=[
            pl.BlockSpec((TB, D), lambda i: (i, 0)),
            pl.BlockSpec((TB, E), lambda i: (i, 0)),
            pl.BlockSpec((D, E), lambda i: (0, 0)),
        ],
        out_specs=pl.BlockSpec((TB, 4), lambda i: (i, 0)),
        out_shape=jax.ShapeDtypeStruct((T, 4), jnp.float32),
        compiler_params=pltpu.CompilerParams(
            dimension_semantics=("parallel",),
        ),
    )(token_hidden, router_logits, egt)

    return out.reshape(T, 2, 2)


# speedup vs baseline: 1.7780x; 1.0391x over previous
import functools

import jax
import jax.numpy as jnp
from jax.experimental import pallas as pl
from jax.experimental.pallas import tpu as pltpu


def _router_kernel(x_ref, r_ref, egt_ref, o_ref):
    x = x_ref[...]
    egt = egt_ref[...]
    dots = jnp.dot(x, egt, preferred_element_type=jnp.float32)
    scores = r_ref[...] + dots

    e_dim = scores.shape[1]
    col = jax.lax.broadcasted_iota(jnp.int32, scores.shape, 1)

    m1 = jnp.max(scores, axis=1, keepdims=True)
    i1 = jnp.min(jnp.where(scores == m1, col, e_dim), axis=1, keepdims=True)
    masked = jnp.where(col == i1, -jnp.inf, scores)
    m2 = jnp.max(masked, axis=1, keepdims=True)
    i2 = jnp.min(jnp.where(masked == m2, col, e_dim), axis=1, keepdims=True)

    e = jnp.exp(m2 - m1)
    s = 1.0 + e
    w0 = 1.0 / s
    w1 = e / s
    o_ref[...] = jnp.concatenate(
        [i1.astype(jnp.float32), w0, i2.astype(jnp.float32), w1], axis=1
    )


def kernel(token_hidden, router_logits, expert_ground, alpha):
    T, D = token_hidden.shape
    E = expert_ground.shape[0]
    TB = 2048
    egt = jnp.float32(alpha) * expert_ground.T

    out = pl.pallas_call(
        _router_kernel,
        grid=(T // TB,),
        in_specs=[
            pl.BlockSpec((TB, D), lambda i: (i, 0)),
            pl.BlockSpec((TB, E), lambda i: (i, 0)),
            pl.BlockSpec((D, E), lambda i: (0, 0)),
        ],
        out_specs=pl.BlockSpec((TB, 4), lambda i: (i, 0)),
        out_shape=jax.ShapeDtypeStruct((T, 4), jnp.float32),
        compiler_params=pltpu.CompilerParams(
            dimension_semantics=("parallel",),
        ),
    )(token_hidden, router_logits, egt)

    return out.reshape(T, 2, 2)
